# baseline (speedup 1.0000x reference)
import functools

import jax
import jax.numpy as jnp
from jax import lax
from jax.experimental import pallas as pl
from jax.experimental.pallas import tpu as pltpu
from jax.experimental.pallas import tpu_sc as plsc

_B, _L, _D = 4096, 200, 64
_N = _B * _L
_NW = 32
_PER_W = _N // _NW
_CHUNK = 320
_NBUF = 4
_NCHUNK = _PER_W // _CHUNK
_NROUND = _NCHUNK // _NBUF


def _emb_body(idx_hbm, table_hbm, out_hbm, idx_v, rows_v, gsem, wsem):
    wid = lax.axis_index("s") * 2 + lax.axis_index("c")
    base = wid * _PER_W
    pltpu.sync_copy(idx_hbm.at[pl.ds(base, _PER_W)], idx_v)

    def _gather_args(c, b):
        off = pl.multiple_of(c * _CHUNK, _CHUNK)
        return (
            table_hbm.at[idx_v.at[pl.ds(off, _CHUNK)]],
            rows_v.at[b],
            gsem.at[b],
        )

    def _write_args(c, b):
        off = pl.multiple_of(c * _CHUNK, _CHUNK)
        return (
            rows_v.at[b],
            out_hbm.at[pl.ds(base + off, _CHUNK), pl.ds(0, _D)],
            wsem.at[b],
        )

    for b in range(_NBUF):
        pltpu.async_copy(*_gather_args(b, b))

    def round_body(g, carry):
        for b in range(_NBUF):
            c = g * _NBUF + b
            pltpu.make_async_copy(*_gather_args(c, b)).wait()
            pltpu.async_copy(*_write_args(c, b))
            pltpu.make_async_copy(*_write_args(c, b)).wait()
            pltpu.async_copy(*_gather_args(c + _NBUF, b))
        return carry

    lax.fori_loop(0, _NROUND - 1, round_body, 0)

    for b in range(_NBUF):
        c = (_NROUND - 1) * _NBUF + b
        pltpu.make_async_copy(*_gather_args(c, b)).wait()
        pltpu.async_copy(*_write_args(c, b))
        pltpu.make_async_copy(*_write_args(c, b)).wait()


_emb = functools.partial(
    pl.kernel,
    out_type=jax.ShapeDtypeStruct((_N, 2 * _D), jnp.float32),
    mesh=plsc.VectorSubcoreMesh(core_axis_name="c", subcore_axis_name="s"),
    scratch_types=[
        pltpu.VMEM((_PER_W,), jnp.int32),
        pltpu.VMEM((_NBUF, _CHUNK, _D), jnp.float32),
        pltpu.SemaphoreType.DMA((_NBUF,)),
        pltpu.SemaphoreType.DMA((_NBUF,)),
    ],
    compiler_params=pltpu.CompilerParams(
        use_tc_tiling_on_sc=False, needs_layout_passes=False
    ),
)(_emb_body)


@jax.jit
def kernel(token_ids, weight):
    idx = token_ids.reshape(_N).astype(jnp.int32)
    out2 = _emb(idx, weight)
    return out2[:, : _D].reshape(_B, _L, _D)


# speedup vs baseline: 1.0015x; 1.0015x over previous
import functools

import jax
import jax.numpy as jnp
from jax import lax
from jax.experimental import pallas as pl
from jax.experimental.pallas import tpu as pltpu
from jax.experimental.pallas import tpu_sc as plsc

_B, _L, _D = 4096, 200, 64
_N = _B * _L
_NW = 32
_PER_W = _N // _NW
_CHUNK = 400
_NBUF = 4
_NCHUNK = _PER_W // _CHUNK
_NROUND = _NCHUNK // _NBUF


def _emb_body(idx_hbm, table_hbm, out_hbm, idx_v, rows_v, gsem, wsem):
    wid = lax.axis_index("s") * 2 + lax.axis_index("c")
    base = wid * _PER_W
    pltpu.sync_copy(idx_hbm.at[pl.ds(base, _PER_W)], idx_v)

    def _gather_args(c, b):
        off = pl.multiple_of(c * _CHUNK, _CHUNK)
        return (
            table_hbm.at[idx_v.at[pl.ds(off, _CHUNK)]],
            rows_v.at[b],
            gsem.at[b],
        )

    def _write_args(c, b):
        off = pl.multiple_of(c * _CHUNK, _CHUNK)
        return (
            rows_v.at[b],
            out_hbm.at[pl.ds(base + off, _CHUNK), pl.ds(0, _D)],
            wsem.at[b],
        )

    for b in range(_NBUF):
        pltpu.async_copy(*_gather_args(b, b))

    def round_body(g, carry):
        for b in range(_NBUF):
            c = g * _NBUF + b
            pltpu.make_async_copy(*_gather_args(c, b)).wait()
            pltpu.async_copy(*_write_args(c, b))
            pltpu.make_async_copy(*_write_args(c, b)).wait()
            pltpu.async_copy(*_gather_args(c + _NBUF, b))
        return carry

    lax.fori_loop(0, _NROUND - 1, round_body, 0)

    for b in range(_NBUF):
        c = (_NROUND - 1) * _NBUF + b
        pltpu.make_async_copy(*_gather_args(c, b)).wait()
        pltpu.async_copy(*_write_args(c, b))
        pltpu.make_async_copy(*_write_args(c, b)).wait()


_emb = functools.partial(
    pl.kernel,
    out_type=jax.ShapeDtypeStruct((_N, 2 * _D), jnp.float32),
    mesh=plsc.VectorSubcoreMesh(core_axis_name="c", subcore_axis_name="s"),
    scratch_types=[
        pltpu.VMEM((_PER_W,), jnp.int32),
        pltpu.VMEM((_NBUF, _CHUNK, _D), jnp.float32),
        pltpu.SemaphoreType.DMA((_NBUF,)),
        pltpu.SemaphoreType.DMA((_NBUF,)),
    ],
    compiler_params=pltpu.CompilerParams(
        use_tc_tiling_on_sc=False, needs_layout_passes=False
    ),
)(_emb_body)


@jax.jit
def kernel(token_ids, weight):
    idx = token_ids.reshape(_N).astype(jnp.int32)
    out2 = _emb(idx, weight)
    return out2[:, : _D].reshape(_B, _L, _D)
